# Initial kernel scaffold; baseline (speedup 1.0000x reference)
import jax
import jax.numpy as jnp
from jax.experimental import pallas as pl


def kernel(features1, edge_index1, features2, edge_index2, W1, b1, W2, b2, Wp1, bp1, Wp2, bp2):
    raise NotImplementedError("write your pallas kernel here")


# speedup vs baseline: 1.8912x; 1.8912x over previous
import functools

import jax
import jax.numpy as jnp
from jax import lax
from jax.experimental import pallas as pl
from jax.experimental.pallas import tpu as pltpu
from jax.experimental.pallas import tpu_sc as plsc

N = 10000
E = 160000
NP = 10240
MP = 2 * NP
NPAD = NP - N
INV_T = 2.0

NTILE = 16
RPT = NP // NTILE
EPT = E // NTILE
K = 80
FCH = 64

BN = 1024
BI = 1024
BJ = 1024


def _sc_edge_loop(table, src_hbm, dst_hbm, acc, idx_s, idx_d, rows, sem, s,
                  dacc=None, ones_v=None):
    def body(i, carry):
        base = s * EPT + i * K
        pltpu.sync_copy(src_hbm.at[pl.ds(base, K)], idx_s)
        pltpu.sync_copy(dst_hbm.at[pl.ds(base, K)], idx_d)
        pltpu.async_copy(table.at[idx_s], rows, sem).wait()
        pltpu.sync_copy(rows, acc.at[idx_d], add=True)
        if dacc is not None:
            pltpu.sync_copy(ones_v, dacc.at[idx_d], add=True)
        return carry

    lax.fori_loop(0, EPT // K, body, 0)


def _sc_agg1_body(x1a, x1b, x2a, x2b, src1, dst1, src2, dst2, z128, ones_hbm,
                  a1a, a1b, a2a, a2b, deg1, deg2,
                  acc, idx_s, idx_d, rows, stg, sem):
    c = lax.axis_index("c")
    s = lax.axis_index("s")

    def zero_acc():
        for k in range(RPT // FCH):
            ch = pl.ds(s * RPT + k * FCH, FCH)
            pltpu.sync_copy(z128.at[ch], stg)
            pltpu.sync_copy(stg, acc.at[ch])

    def flush_acc(out_h):
        for k in range(RPT // FCH):
            ch = pl.ds(s * RPT + k * FCH, FCH)
            pltpu.sync_copy(acc.at[ch], stg)
            pltpu.sync_copy(stg, out_h.at[ch])

    def deg_loop(dst_h):
        def body(i, carry):
            base = s * EPT + i * K
            pltpu.sync_copy(dst_h.at[pl.ds(base, K)], idx_d)
            pltpu.sync_copy(rows, acc.at[idx_d], add=True)
            return carry
        lax.fori_loop(0, EPT // K, body, 0)

    zero_acc()
    plsc.subcore_barrier()

    for ta, tb, out in ((x1a, x2a, (a1a, a2a)), (x1b, x2b, (a1b, a2b))):
        @pl.when(c == 0)
        def _():
            _sc_edge_loop(ta, src1, dst1, acc, idx_s, idx_d, rows, sem, s)

        @pl.when(c == 1)
        def _():
            _sc_edge_loop(tb, src2, dst2, acc, idx_s, idx_d, rows, sem, s)

        plsc.subcore_barrier()

        @pl.when(c == 0)
        def _():
            flush_acc(out[0])

        @pl.when(c == 1)
        def _():
            flush_acc(out[1])

        zero_acc()
        plsc.subcore_barrier()

    pltpu.sync_copy(ones_hbm, rows)

    @pl.when(c == 0)
    def _():
        deg_loop(dst1)

    @pl.when(c == 1)
    def _():
        deg_loop(dst2)

    plsc.subcore_barrier()

    @pl.when(c == 0)
    def _():
        flush_acc(deg1)

    @pl.when(c == 1)
    def _():
        flush_acc(deg2)


def _sc_agg2_body(h10, h11, h12, h13, h20, h21, h22, h23,
                  src1, dst1, src2, dst2, z128,
                  b10, b11, b12, b13, b20, b21, b22, b23,
                  acc, idx_s, idx_d, rows, stg, sem):
    c = lax.axis_index("c")
    s = lax.axis_index("s")
    stripe = pl.ds(s * RPT, RPT)

    def zero_acc():
        for k in range(RPT // FCH):
            ch = pl.ds(s * RPT + k * FCH, FCH)
            pltpu.sync_copy(z128.at[ch], stg)
            pltpu.sync_copy(stg, acc.at[ch])

    def flush_acc(out_h):
        for k in range(RPT // FCH):
            ch = pl.ds(s * RPT + k * FCH, FCH)
            pltpu.sync_copy(acc.at[ch], stg)
            pltpu.sync_copy(stg, out_h.at[ch])

    zero_acc()
    plsc.subcore_barrier()

    rounds = (
        (h10, h11, src1, dst1, b10, b11),
        (h12, h13, src1, dst1, b12, b13),
        (h20, h21, src2, dst2, b20, b21),
        (h22, h23, src2, dst2, b22, b23),
    )
    for r, (ta, tb, srcr, dstr, oa, ob) in enumerate(rounds):
        @pl.when(c == 0)
        def _():
            _sc_edge_loop(ta, srcr, dstr, acc, idx_s, idx_d, rows, sem, s)

        @pl.when(c == 1)
        def _():
            _sc_edge_loop(tb, srcr, dstr, acc, idx_s, idx_d, rows, sem, s)

        plsc.subcore_barrier()

        @pl.when(c == 0)
        def _():
            flush_acc(oa)

        @pl.when(c == 1)
        def _():
            flush_acc(ob)

        if r != 3:
            zero_acc()
            plsc.subcore_barrier()


@functools.cache
def _sc_kernels():
    mesh = plsc.VectorSubcoreMesh(core_axis_name="c", subcore_axis_name="s")
    f128 = jax.ShapeDtypeStruct((NP, 128), jnp.float32)
    agg1 = pl.kernel(
        _sc_agg1_body,
        out_type=(f128,) * 6,
        mesh=mesh,
        scratch_types=[
            pltpu.VMEM_SHARED((NP, 128), jnp.float32),
            pltpu.VMEM((K,), jnp.int32),
            pltpu.VMEM((K,), jnp.int32),
            pltpu.VMEM((K, 128), jnp.float32),
            pltpu.VMEM((FCH, 128), jnp.float32),
            pltpu.SemaphoreType.DMA,
        ],
    )
    agg2 = pl.kernel(
        _sc_agg2_body,
        out_type=(f128,) * 8,
        mesh=mesh,
        scratch_types=[
            pltpu.VMEM_SHARED((NP, 128), jnp.float32),
            pltpu.VMEM((K,), jnp.int32),
            pltpu.VMEM((K,), jnp.int32),
            pltpu.VMEM((K, 128), jnp.float32),
            pltpu.VMEM((FCH, 128), jnp.float32),
            pltpu.SemaphoreType.DMA,
        ],
    )
    return agg1, agg2


def _l1_body(agga_ref, aggb_ref, x_ref, deg_ref, w1_ref, b1_ref, h1t_ref):
    agg = jnp.concatenate([agga_ref[...], aggb_ref[...]], axis=1)
    u = (agg + x_ref[...]) / (deg_ref[:, :1] + 1.0)
    h = lax.dot_general(u, w1_ref[...], (((1,), (1,)), ((), ())))
    h = jnp.maximum(h + b1_ref[...], 0.0)
    for q in range(4):
        h1t_ref[q] = h[:, 128 * q:128 * (q + 1)]


def _l2_body(g0_ref, g1_ref, g2_ref, g3_ref, h1t_ref, deg_ref, w2_ref, b2_ref,
             wp1_ref, bp1_ref, wp2_ref, bp2_ref, n_ref, nbf_ref):
    i = pl.program_id(0)
    agg = jnp.concatenate(
        [g0_ref[...], g1_ref[...], g2_ref[...], g3_ref[...]], axis=1)
    h1 = jnp.concatenate([h1t_ref[q] for q in range(4)], axis=1)
    u = (agg + h1) / (deg_ref[:, :1] + 1.0)
    h2 = lax.dot_general(u, w2_ref[...], (((1,), (1,)), ((), ())))
    h2 = jnp.maximum(h2 + b2_ref[...], 0.0)
    e = lax.dot_general(h2, wp1_ref[...], (((1,), (1,)), ((), ())))
    e = e + bp1_ref[...]
    e = jnp.where(e > 0.0, e, jnp.exp(e) - 1.0)
    p = lax.dot_general(e, wp2_ref[...], (((1,), (1,)), ((), ())))
    p = p + bp2_ref[...]
    nrm = jnp.sqrt(jnp.sum(p * p, axis=1, keepdims=True))
    n = p / jnp.maximum(nrm, 1e-12)
    row = i * BN + lax.broadcasted_iota(jnp.int32, (BN, 1), 0)
    n = jnp.where(row < N, n, 0.0)
    n_ref[...] = n
    nbf_ref[...] = n.astype(jnp.bfloat16)


def _loss_body(q_ref, k_ref, s_ref):
    j = pl.program_id(1)

    @pl.when(j == 0)
    def _():
        s_ref[...] = jnp.zeros_like(s_ref)

    z = lax.dot_general(q_ref[...], k_ref[...], (((1,), (1,)), ((), ())),
                        preferred_element_type=jnp.float32)
    s_ref[...] += jnp.sum(jnp.exp(z * INV_T), axis=1, keepdims=True)


def _fin_body(n1_ref, n2_ref, s1_ref, s2_ref, o_ref):
    i = pl.program_id(0)

    @pl.when(i == 0)
    def _():
        o_ref[...] = jnp.zeros_like(o_ref)

    n1 = n1_ref[...]
    n2 = n2_ref[...]
    d = jnp.sum(n1 * n2, axis=1, keepdims=True)
    q1 = jnp.sum(n1 * n1, axis=1, keepdims=True)
    q2 = jnp.sum(n2 * n2, axis=1, keepdims=True)
    den1 = s1_ref[...] - float(2 * NPAD) - jnp.exp(q1 * INV_T)
    den2 = s2_ref[...] - float(2 * NPAD) - jnp.exp(q2 * INV_T)
    v = -d * INV_T + 0.5 * (jnp.log(den1) + jnp.log(den2))
    row = i * BN + lax.broadcasted_iota(jnp.int32, (BN, 1), 0)
    v = jnp.where(row < N, v, 0.0)
    o_ref[...] = o_ref[...] + jnp.sum(v) / float(N)


_l1_call = pl.pallas_call(
    _l1_body,
    grid=(NP // BN,),
    in_specs=[
        pl.BlockSpec((BN, 128), lambda i: (i, 0)),
        pl.BlockSpec((BN, 128), lambda i: (i, 0)),
        pl.BlockSpec((BN, 256), lambda i: (i, 0)),
        pl.BlockSpec((BN, 128), lambda i: (i, 0)),
        pl.BlockSpec((512, 256), lambda i: (0, 0)),
        pl.BlockSpec((1, 512), lambda i: (0, 0)),
    ],
    out_specs=pl.BlockSpec((4, BN, 128), lambda i: (0, i, 0)),
    out_shape=jax.ShapeDtypeStruct((4, NP, 128), jnp.float32),
)

_l2_call = pl.pallas_call(
    _l2_body,
    grid=(NP // BN,),
    in_specs=[
        pl.BlockSpec((BN, 128), lambda i: (i, 0)),
        pl.BlockSpec((BN, 128), lambda i: (i, 0)),
        pl.BlockSpec((BN, 128), lambda i: (i, 0)),
        pl.BlockSpec((BN, 128), lambda i: (i, 0)),
        pl.BlockSpec((4, BN, 128), lambda i: (0, i, 0)),
        pl.BlockSpec((BN, 128), lambda i: (i, 0)),
        pl.BlockSpec((256, 512), lambda i: (0, 0)),
        pl.BlockSpec((1, 256), lambda i: (0, 0)),
        pl.BlockSpec((128, 256), lambda i: (0, 0)),
        pl.BlockSpec((1, 128), lambda i: (0, 0)),
        pl.BlockSpec((256, 128), lambda i: (0, 0)),
        pl.BlockSpec((1, 256), lambda i: (0, 0)),
    ],
    out_specs=[
        pl.BlockSpec((BN, 256), lambda i: (i, 0)),
        pl.BlockSpec((BN, 256), lambda i: (i, 0)),
    ],
    out_shape=[
        jax.ShapeDtypeStruct((NP, 256), jnp.float32),
        jax.ShapeDtypeStruct((NP, 256), jnp.bfloat16),
    ],
)

_loss_call = pl.pallas_call(
    _loss_body,
    grid=(MP // BI, MP // BJ),
    in_specs=[
        pl.BlockSpec((BI, 256), lambda i, j: (i, 0)),
        pl.BlockSpec((BJ, 256), lambda i, j: (j, 0)),
    ],
    out_specs=pl.BlockSpec((BI, 1), lambda i, j: (i, 0)),
    out_shape=jax.ShapeDtypeStruct((MP, 1), jnp.float32),
)

_fin_call = pl.pallas_call(
    _fin_body,
    grid=(NP // BN,),
    in_specs=[
        pl.BlockSpec((BN, 256), lambda i: (i, 0)),
        pl.BlockSpec((BN, 256), lambda i: (i, 0)),
        pl.BlockSpec((BN, 1), lambda i: (i, 0)),
        pl.BlockSpec((BN, 1), lambda i: (i, 0)),
    ],
    out_specs=pl.BlockSpec((1, 1), lambda i: (0, 0)),
    out_shape=jax.ShapeDtypeStruct((1, 1), jnp.float32),
)


@jax.jit
def kernel(features1, edge_index1, features2, edge_index2,
           W1, b1, W2, b2, Wp1, bp1, Wp2, bp2):
    x1 = jnp.pad(features1, ((0, NPAD), (0, 0)))
    x2 = jnp.pad(features2, ((0, NPAD), (0, 0)))
    src1 = edge_index1[0].astype(jnp.int32)
    dst1 = edge_index1[1].astype(jnp.int32)
    src2 = edge_index2[0].astype(jnp.int32)
    dst2 = edge_index2[1].astype(jnp.int32)
    z128 = jnp.zeros((NP, 128), jnp.float32)
    ones = jnp.ones((K, 128), jnp.float32)

    sc_agg1, sc_agg2 = _sc_kernels()
    a1a, a1b, a2a, a2b, deg1, deg2 = sc_agg1(
        x1[:, :128], x1[:, 128:], x2[:, :128], x2[:, 128:],
        src1, dst1, src2, dst2, z128, ones)

    h1t_1 = _l1_call(a1a, a1b, x1, deg1, W1, b1.reshape(1, -1))
    h1t_2 = _l1_call(a2a, a2b, x2, deg2, W1, b1.reshape(1, -1))

    g1 = sc_agg2(h1t_1[0], h1t_1[1], h1t_1[2], h1t_1[3],
                 h1t_2[0], h1t_2[1], h1t_2[2], h1t_2[3],
                 src1, dst1, src2, dst2, z128)

    n1, n1bf = _l2_call(g1[0], g1[1], g1[2], g1[3], h1t_1, deg1,
                        W2, b2.reshape(1, -1),
                        Wp1, bp1.reshape(1, -1), Wp2, bp2.reshape(1, -1))
    n2, n2bf = _l2_call(g1[4], g1[5], g1[6], g1[7], h1t_2, deg2,
                        W2, b2.reshape(1, -1),
                        Wp1, bp1.reshape(1, -1), Wp2, bp2.reshape(1, -1))

    abf = jnp.concatenate([n1bf, n2bf], axis=0)
    s = _loss_call(abf, abf)

    out = _fin_call(n1, n2, s[:NP], s[NP:])
    return out[0, 0]


# speedup vs baseline: 2.6674x; 1.4104x over previous
import functools

import jax
import jax.numpy as jnp
from jax import lax
from jax.experimental import pallas as pl
from jax.experimental.pallas import tpu as pltpu
from jax.experimental.pallas import tpu_sc as plsc

N = 10000
E = 160000
NP = 10240
MP = 2 * NP
NPAD = NP - N
INV_T = 2.0

NTILE = 16
RPT = NP // NTILE
EPT = E // NTILE
K = 80
FCH = 64

BN = 1024
BI = 1024
BJ = 1024


NCH = EPT // K


def _sc_edge_loop(table, src_hbm, dst_hbm, drain_hbm, acc, bufs, s):
    (isa, ida, rowsa, sema), (isb, idb, rowsb, semb) = bufs

    def start_gather(ch, idx_s, rows, sem):
        pltpu.sync_copy(src_hbm.at[pl.ds(s * EPT + ch * K, K)], idx_s)
        pltpu.async_copy(table.at[idx_s], rows, sem)

    def finish_scatter(ch, idx_d, rows, sem):
        pltpu.sync_copy(dst_hbm.at[pl.ds(s * EPT + ch * K, K)], idx_d)
        pltpu.make_async_copy(drain_hbm.at[pl.ds(0, K)], rows, sem).wait()
        pltpu.sync_copy(rows, acc.at[idx_d], add=True)

    start_gather(0, isa, rowsa, sema)

    def body(j, carry):
        c0 = 2 * j
        start_gather(c0 + 1, isb, rowsb, semb)
        finish_scatter(c0, ida, rowsa, sema)
        start_gather(c0 + 2, isa, rowsa, sema)
        finish_scatter(c0 + 1, idb, rowsb, semb)
        return carry

    lax.fori_loop(0, (NCH - 1) // 2, body, 0)
    finish_scatter(NCH - 1, ida, rowsa, sema)


def _sc_agg1_body(x1a, x1b, x2a, x2b, src1, dst1, src2, dst2, z128, ones_hbm,
                  a1a, a1b, a2a, a2b, deg1, deg2,
                  acc, isa, ida, rowsa, sema, isb, idb, rowsb, semb, stg):
    bufs = ((isa, ida, rowsa, sema), (isb, idb, rowsb, semb))
    c = lax.axis_index("c")
    s = lax.axis_index("s")

    def zero_acc():
        for k in range(RPT // FCH):
            ch = pl.ds(s * RPT + k * FCH, FCH)
            pltpu.sync_copy(z128.at[ch], stg)
            pltpu.sync_copy(stg, acc.at[ch])

    def flush_acc(out_h):
        for k in range(RPT // FCH):
            ch = pl.ds(s * RPT + k * FCH, FCH)
            pltpu.sync_copy(acc.at[ch], stg)
            pltpu.sync_copy(stg, out_h.at[ch])

    def deg_loop(dst_h):
        def body(i, carry):
            base = s * EPT + i * K
            pltpu.sync_copy(dst_h.at[pl.ds(base, K)], ida)
            pltpu.sync_copy(rowsa, acc.at[ida], add=True)
            return carry
        lax.fori_loop(0, EPT // K, body, 0)

    zero_acc()
    plsc.subcore_barrier()

    for ta, tb, out in ((x1a, x2a, (a1a, a2a)), (x1b, x2b, (a1b, a2b))):
        @pl.when(c == 0)
        def _():
            _sc_edge_loop(ta, src1, dst1, z128, acc, bufs, s)

        @pl.when(c == 1)
        def _():
            _sc_edge_loop(tb, src2, dst2, z128, acc, bufs, s)

        plsc.subcore_barrier()

        @pl.when(c == 0)
        def _():
            flush_acc(out[0])

        @pl.when(c == 1)
        def _():
            flush_acc(out[1])

        zero_acc()
        plsc.subcore_barrier()

    pltpu.sync_copy(ones_hbm, rowsa)

    @pl.when(c == 0)
    def _():
        deg_loop(dst1)

    @pl.when(c == 1)
    def _():
        deg_loop(dst2)

    plsc.subcore_barrier()

    @pl.when(c == 0)
    def _():
        flush_acc(deg1)

    @pl.when(c == 1)
    def _():
        flush_acc(deg2)


def _sc_agg2_body(h10, h11, h12, h13, h20, h21, h22, h23,
                  src1, dst1, src2, dst2, z128,
                  b10, b11, b12, b13, b20, b21, b22, b23,
                  acc, isa, ida, rowsa, sema, isb, idb, rowsb, semb, stg):
    bufs = ((isa, ida, rowsa, sema), (isb, idb, rowsb, semb))
    c = lax.axis_index("c")
    s = lax.axis_index("s")

    def zero_acc():
        for k in range(RPT // FCH):
            ch = pl.ds(s * RPT + k * FCH, FCH)
            pltpu.sync_copy(z128.at[ch], stg)
            pltpu.sync_copy(stg, acc.at[ch])

    def flush_acc(out_h):
        for k in range(RPT // FCH):
            ch = pl.ds(s * RPT + k * FCH, FCH)
            pltpu.sync_copy(acc.at[ch], stg)
            pltpu.sync_copy(stg, out_h.at[ch])

    zero_acc()
    plsc.subcore_barrier()

    rounds = (
        (h10, h11, src1, dst1, b10, b11),
        (h12, h13, src1, dst1, b12, b13),
        (h20, h21, src2, dst2, b20, b21),
        (h22, h23, src2, dst2, b22, b23),
    )
    for r, (ta, tb, srcr, dstr, oa, ob) in enumerate(rounds):
        @pl.when(c == 0)
        def _():
            _sc_edge_loop(ta, srcr, dstr, z128, acc, bufs, s)

        @pl.when(c == 1)
        def _():
            _sc_edge_loop(tb, srcr, dstr, z128, acc, bufs, s)

        plsc.subcore_barrier()

        @pl.when(c == 0)
        def _():
            flush_acc(oa)

        @pl.when(c == 1)
        def _():
            flush_acc(ob)

        if r != 3:
            zero_acc()
            plsc.subcore_barrier()


@functools.cache
def _sc_kernels():
    mesh = plsc.VectorSubcoreMesh(core_axis_name="c", subcore_axis_name="s")
    f128 = jax.ShapeDtypeStruct((NP, 128), jnp.float32)
    agg1 = pl.kernel(
        _sc_agg1_body,
        out_type=(f128,) * 6,
        mesh=mesh,
        scratch_types=[
            pltpu.VMEM_SHARED((NP, 128), jnp.float32),
            pltpu.VMEM((K,), jnp.int32),
            pltpu.VMEM((K,), jnp.int32),
            pltpu.VMEM((K, 128), jnp.float32),
            pltpu.SemaphoreType.DMA,
            pltpu.VMEM((K,), jnp.int32),
            pltpu.VMEM((K,), jnp.int32),
            pltpu.VMEM((K, 128), jnp.float32),
            pltpu.SemaphoreType.DMA,
            pltpu.VMEM((FCH, 128), jnp.float32),
        ],
    )
    agg2 = pl.kernel(
        _sc_agg2_body,
        out_type=(f128,) * 8,
        mesh=mesh,
        scratch_types=[
            pltpu.VMEM_SHARED((NP, 128), jnp.float32),
            pltpu.VMEM((K,), jnp.int32),
            pltpu.VMEM((K,), jnp.int32),
            pltpu.VMEM((K, 128), jnp.float32),
            pltpu.SemaphoreType.DMA,
            pltpu.VMEM((K,), jnp.int32),
            pltpu.VMEM((K,), jnp.int32),
            pltpu.VMEM((K, 128), jnp.float32),
            pltpu.SemaphoreType.DMA,
            pltpu.VMEM((FCH, 128), jnp.float32),
        ],
    )
    return agg1, agg2


def _l1_body(agga_ref, aggb_ref, x_ref, deg_ref, w1_ref, b1_ref, h1t_ref):
    agg = jnp.concatenate([agga_ref[...], aggb_ref[...]], axis=1)
    u = (agg + x_ref[...]) / (deg_ref[:, :1] + 1.0)
    h = lax.dot_general(u, w1_ref[...], (((1,), (1,)), ((), ())))
    h = jnp.maximum(h + b1_ref[...], 0.0)
    for q in range(4):
        h1t_ref[q] = h[:, 128 * q:128 * (q + 1)]


def _l2_body(g0_ref, g1_ref, g2_ref, g3_ref, h1t_ref, deg_ref, w2_ref, b2_ref,
             wp1_ref, bp1_ref, wp2_ref, bp2_ref, n_ref, nbf_ref):
    i = pl.program_id(0)
    agg = jnp.concatenate(
        [g0_ref[...], g1_ref[...], g2_ref[...], g3_ref[...]], axis=1)
    h1 = jnp.concatenate([h1t_ref[q] for q in range(4)], axis=1)
    u = (agg + h1) / (deg_ref[:, :1] + 1.0)
    h2 = lax.dot_general(u, w2_ref[...], (((1,), (1,)), ((), ())))
    h2 = jnp.maximum(h2 + b2_ref[...], 0.0)
    e = lax.dot_general(h2, wp1_ref[...], (((1,), (1,)), ((), ())))
    e = e + bp1_ref[...]
    e = jnp.where(e > 0.0, e, jnp.exp(e) - 1.0)
    p = lax.dot_general(e, wp2_ref[...], (((1,), (1,)), ((), ())))
    p = p + bp2_ref[...]
    nrm = jnp.sqrt(jnp.sum(p * p, axis=1, keepdims=True))
    n = p / jnp.maximum(nrm, 1e-12)
    row = i * BN + lax.broadcasted_iota(jnp.int32, (BN, 1), 0)
    n = jnp.where(row < N, n, 0.0)
    n_ref[...] = n
    nbf_ref[...] = n.astype(jnp.bfloat16)


def _loss_body(q_ref, k_ref, s_ref):
    j = pl.program_id(1)

    @pl.when(j == 0)
    def _():
        s_ref[...] = jnp.zeros_like(s_ref)

    z = lax.dot_general(q_ref[...], k_ref[...], (((1,), (1,)), ((), ())),
                        preferred_element_type=jnp.float32)
    s_ref[...] += jnp.sum(jnp.exp(z * INV_T), axis=1, keepdims=True)


def _fin_body(n1_ref, n2_ref, s1_ref, s2_ref, o_ref):
    i = pl.program_id(0)

    @pl.when(i == 0)
    def _():
        o_ref[...] = jnp.zeros_like(o_ref)

    n1 = n1_ref[...]
    n2 = n2_ref[...]
    d = jnp.sum(n1 * n2, axis=1, keepdims=True)
    q1 = jnp.sum(n1 * n1, axis=1, keepdims=True)
    q2 = jnp.sum(n2 * n2, axis=1, keepdims=True)
    den1 = s1_ref[...] - float(2 * NPAD) - jnp.exp(q1 * INV_T)
    den2 = s2_ref[...] - float(2 * NPAD) - jnp.exp(q2 * INV_T)
    v = -d * INV_T + 0.5 * (jnp.log(den1) + jnp.log(den2))
    row = i * BN + lax.broadcasted_iota(jnp.int32, (BN, 1), 0)
    v = jnp.where(row < N, v, 0.0)
    o_ref[...] = o_ref[...] + jnp.sum(v) / float(N)


_l1_call = pl.pallas_call(
    _l1_body,
    grid=(NP // BN,),
    in_specs=[
        pl.BlockSpec((BN, 128), lambda i: (i, 0)),
        pl.BlockSpec((BN, 128), lambda i: (i, 0)),
        pl.BlockSpec((BN, 256), lambda i: (i, 0)),
        pl.BlockSpec((BN, 128), lambda i: (i, 0)),
        pl.BlockSpec((512, 256), lambda i: (0, 0)),
        pl.BlockSpec((1, 512), lambda i: (0, 0)),
    ],
    out_specs=pl.BlockSpec((4, BN, 128), lambda i: (0, i, 0)),
    out_shape=jax.ShapeDtypeStruct((4, NP, 128), jnp.float32),
)

_l2_call = pl.pallas_call(
    _l2_body,
    grid=(NP // BN,),
    in_specs=[
        pl.BlockSpec((BN, 128), lambda i: (i, 0)),
        pl.BlockSpec((BN, 128), lambda i: (i, 0)),
        pl.BlockSpec((BN, 128), lambda i: (i, 0)),
        pl.BlockSpec((BN, 128), lambda i: (i, 0)),
        pl.BlockSpec((4, BN, 128), lambda i: (0, i, 0)),
        pl.BlockSpec((BN, 128), lambda i: (i, 0)),
        pl.BlockSpec((256, 512), lambda i: (0, 0)),
        pl.BlockSpec((1, 256), lambda i: (0, 0)),
        pl.BlockSpec((128, 256), lambda i: (0, 0)),
        pl.BlockSpec((1, 128), lambda i: (0, 0)),
        pl.BlockSpec((256, 128), lambda i: (0, 0)),
        pl.BlockSpec((1, 256), lambda i: (0, 0)),
    ],
    out_specs=[
        pl.BlockSpec((BN, 256), lambda i: (i, 0)),
        pl.BlockSpec((BN, 256), lambda i: (i, 0)),
    ],
    out_shape=[
        jax.ShapeDtypeStruct((NP, 256), jnp.float32),
        jax.ShapeDtypeStruct((NP, 256), jnp.bfloat16),
    ],
)

_loss_call = pl.pallas_call(
    _loss_body,
    grid=(MP // BI, MP // BJ),
    in_specs=[
        pl.BlockSpec((BI, 256), lambda i, j: (i, 0)),
        pl.BlockSpec((BJ, 256), lambda i, j: (j, 0)),
    ],
    out_specs=pl.BlockSpec((BI, 1), lambda i, j: (i, 0)),
    out_shape=jax.ShapeDtypeStruct((MP, 1), jnp.float32),
)

_fin_call = pl.pallas_call(
    _fin_body,
    grid=(NP // BN,),
    in_specs=[
        pl.BlockSpec((BN, 256), lambda i: (i, 0)),
        pl.BlockSpec((BN, 256), lambda i: (i, 0)),
        pl.BlockSpec((BN, 1), lambda i: (i, 0)),
        pl.BlockSpec((BN, 1), lambda i: (i, 0)),
    ],
    out_specs=pl.BlockSpec((1, 1), lambda i: (0, 0)),
    out_shape=jax.ShapeDtypeStruct((1, 1), jnp.float32),
)


@jax.jit
def kernel(features1, edge_index1, features2, edge_index2,
           W1, b1, W2, b2, Wp1, bp1, Wp2, bp2):
    x1 = jnp.pad(features1, ((0, NPAD), (0, 0)))
    x2 = jnp.pad(features2, ((0, NPAD), (0, 0)))
    src1 = edge_index1[0].astype(jnp.int32)
    dst1 = edge_index1[1].astype(jnp.int32)
    src2 = edge_index2[0].astype(jnp.int32)
    dst2 = edge_index2[1].astype(jnp.int32)
    z128 = jnp.zeros((NP, 128), jnp.float32)
    ones = jnp.ones((K, 128), jnp.float32)

    sc_agg1, sc_agg2 = _sc_kernels()
    a1a, a1b, a2a, a2b, deg1, deg2 = sc_agg1(
        x1[:, :128], x1[:, 128:], x2[:, :128], x2[:, 128:],
        src1, dst1, src2, dst2, z128, ones)

    h1t_1 = _l1_call(a1a, a1b, x1, deg1, W1, b1.reshape(1, -1))
    h1t_2 = _l1_call(a2a, a2b, x2, deg2, W1, b1.reshape(1, -1))

    g1 = sc_agg2(h1t_1[0], h1t_1[1], h1t_1[2], h1t_1[3],
                 h1t_2[0], h1t_2[1], h1t_2[2], h1t_2[3],
                 src1, dst1, src2, dst2, z128)

    n1, n1bf = _l2_call(g1[0], g1[1], g1[2], g1[3], h1t_1, deg1,
                        W2, b2.reshape(1, -1),
                        Wp1, bp1.reshape(1, -1), Wp2, bp2.reshape(1, -1))
    n2, n2bf = _l2_call(g1[4], g1[5], g1[6], g1[7], h1t_2, deg2,
                        W2, b2.reshape(1, -1),
                        Wp1, bp1.reshape(1, -1), Wp2, bp2.reshape(1, -1))

    abf = jnp.concatenate([n1bf, n2bf], axis=0)
    s = _loss_call(abf, abf)

    out = _fin_call(n1, n2, s[:NP], s[NP:])
    return out[0, 0]


# speedup vs baseline: 2.7406x; 1.0274x over previous
import functools

import jax
import jax.numpy as jnp
from jax import lax
from jax.experimental import pallas as pl
from jax.experimental.pallas import tpu as pltpu
from jax.experimental.pallas import tpu_sc as plsc

N = 10000
E = 160000
NP = 10240
MP = 2 * NP
NPAD = NP - N
INV_T = 2.0

NTILE = 16
RPT = NP // NTILE
EPT = E // NTILE
K = 80
FCH = 64

BN = 1024
BI = 1024
BJ = 1024


NCH = EPT // K


def _sc_edge_loop(table, src_hbm, dst_hbm, drain_hbm, acc, bufs, s):
    (isa, ida, rowsa, sema), (isb, idb, rowsb, semb) = bufs

    def start_gather(ch, idx_s, rows, sem):
        pltpu.sync_copy(src_hbm.at[pl.ds(s * EPT + ch * K, K)], idx_s)
        pltpu.async_copy(table.at[idx_s], rows, sem)

    def finish_scatter(ch, idx_d, rows, sem):
        pltpu.sync_copy(dst_hbm.at[pl.ds(s * EPT + ch * K, K)], idx_d)
        pltpu.make_async_copy(drain_hbm.at[pl.ds(0, K)], rows, sem).wait()
        pltpu.sync_copy(rows, acc.at[idx_d], add=True)

    start_gather(0, isa, rowsa, sema)

    def body(j, carry):
        c0 = 2 * j
        start_gather(c0 + 1, isb, rowsb, semb)
        finish_scatter(c0, ida, rowsa, sema)
        start_gather(c0 + 2, isa, rowsa, sema)
        finish_scatter(c0 + 1, idb, rowsb, semb)
        return carry

    lax.fori_loop(0, (NCH - 1) // 2, body, 0)
    finish_scatter(NCH - 1, ida, rowsa, sema)


def _sc_agg1_body(x1a, x1b, x2a, x2b, src1, dst1, src2, dst2, z128, ones_hbm,
                  a1a, a1b, a2a, a2b, deg1, deg2,
                  acc, isa, ida, rowsa, sema, isb, idb, rowsb, semb, stg):
    bufs = ((isa, ida, rowsa, sema), (isb, idb, rowsb, semb))
    c = lax.axis_index("c")
    s = lax.axis_index("s")

    def zero_acc():
        for k in range(RPT // FCH):
            ch = pl.ds(s * RPT + k * FCH, FCH)
            pltpu.sync_copy(z128.at[ch], stg)
            pltpu.sync_copy(stg, acc.at[ch])

    def flush_acc(out_h):
        for k in range(RPT // FCH):
            ch = pl.ds(s * RPT + k * FCH, FCH)
            pltpu.sync_copy(acc.at[ch], stg)
            pltpu.sync_copy(stg, out_h.at[ch])

    def deg_loop(dst_h):
        def body(i, carry):
            base = s * EPT + i * K
            pltpu.sync_copy(dst_h.at[pl.ds(base, K)], ida)
            pltpu.sync_copy(rowsa, acc.at[ida], add=True)
            return carry
        lax.fori_loop(0, EPT // K, body, 0)

    zero_acc()
    plsc.subcore_barrier()

    for ta, tb, out in ((x1a, x2a, (a1a, a2a)), (x1b, x2b, (a1b, a2b))):
        @pl.when(c == 0)
        def _():
            _sc_edge_loop(ta, src1, dst1, z128, acc, bufs, s)

        @pl.when(c == 1)
        def _():
            _sc_edge_loop(tb, src2, dst2, z128, acc, bufs, s)

        plsc.subcore_barrier()

        @pl.when(c == 0)
        def _():
            flush_acc(out[0])

        @pl.when(c == 1)
        def _():
            flush_acc(out[1])

        zero_acc()
        plsc.subcore_barrier()

    pltpu.sync_copy(ones_hbm, rowsa)

    @pl.when(c == 0)
    def _():
        deg_loop(dst1)

    @pl.when(c == 1)
    def _():
        deg_loop(dst2)

    plsc.subcore_barrier()

    @pl.when(c == 0)
    def _():
        flush_acc(deg1)

    @pl.when(c == 1)
    def _():
        flush_acc(deg2)


def _sc_agg2_body(h0, h1, h2, h3, srcr, dstr, z128,
                  b0, b1, b2, b3,
                  acc, isa, ida, rowsa, sema, isb, idb, rowsb, semb, stg):
    bufs = ((isa, ida, rowsa, sema), (isb, idb, rowsb, semb))
    c = lax.axis_index("c")
    s = lax.axis_index("s")

    def zero_acc():
        for k in range(RPT // FCH):
            ch = pl.ds(s * RPT + k * FCH, FCH)
            pltpu.sync_copy(z128.at[ch], stg)
            pltpu.sync_copy(stg, acc.at[ch])

    def flush_acc(out_h):
        for k in range(RPT // FCH):
            ch = pl.ds(s * RPT + k * FCH, FCH)
            pltpu.sync_copy(acc.at[ch], stg)
            pltpu.sync_copy(stg, out_h.at[ch])

    zero_acc()
    plsc.subcore_barrier()

    for r, (ta, tb, oa, ob) in enumerate(((h0, h1, b0, b1), (h2, h3, b2, b3))):
        @pl.when(c == 0)
        def _():
            _sc_edge_loop(ta, srcr, dstr, z128, acc, bufs, s)

        @pl.when(c == 1)
        def _():
            _sc_edge_loop(tb, srcr, dstr, z128, acc, bufs, s)

        plsc.subcore_barrier()

        @pl.when(c == 0)
        def _():
            flush_acc(oa)

        @pl.when(c == 1)
        def _():
            flush_acc(ob)

        if r != 1:
            zero_acc()
            plsc.subcore_barrier()


@functools.cache
def _sc_kernels():
    mesh = plsc.VectorSubcoreMesh(core_axis_name="c", subcore_axis_name="s")
    f128 = jax.ShapeDtypeStruct((NP, 128), jnp.float32)
    agg1 = pl.kernel(
        _sc_agg1_body,
        out_type=(f128,) * 6,
        mesh=mesh,
        scratch_types=[
            pltpu.VMEM_SHARED((NP, 128), jnp.float32),
            pltpu.VMEM((K,), jnp.int32),
            pltpu.VMEM((K,), jnp.int32),
            pltpu.VMEM((K, 128), jnp.float32),
            pltpu.SemaphoreType.DMA,
            pltpu.VMEM((K,), jnp.int32),
            pltpu.VMEM((K,), jnp.int32),
            pltpu.VMEM((K, 128), jnp.float32),
            pltpu.SemaphoreType.DMA,
            pltpu.VMEM((FCH, 128), jnp.float32),
        ],
    )
    agg2 = pl.kernel(
        _sc_agg2_body,
        out_type=(f128,) * 4,
        mesh=mesh,
        scratch_types=[
            pltpu.VMEM_SHARED((NP, 128), jnp.float32),
            pltpu.VMEM((K,), jnp.int32),
            pltpu.VMEM((K,), jnp.int32),
            pltpu.VMEM((K, 128), jnp.float32),
            pltpu.SemaphoreType.DMA,
            pltpu.VMEM((K,), jnp.int32),
            pltpu.VMEM((K,), jnp.int32),
            pltpu.VMEM((K, 128), jnp.float32),
            pltpu.SemaphoreType.DMA,
            pltpu.VMEM((FCH, 128), jnp.float32),
        ],
    )
    return agg1, agg2


def _l1_body(agga_ref, aggb_ref, x_ref, deg_ref, w1_ref, b1_ref, h1t_ref):
    agg = jnp.concatenate([agga_ref[...], aggb_ref[...]], axis=1)
    u = (agg + x_ref[...]) / (deg_ref[:, :1] + 1.0)
    h = lax.dot_general(u, w1_ref[...], (((1,), (1,)), ((), ())))
    h = jnp.maximum(h + b1_ref[...], 0.0)
    for q in range(4):
        h1t_ref[q] = h[:, 128 * q:128 * (q + 1)]


def _l2_body(g0_ref, g1_ref, g2_ref, g3_ref, h1t_ref, deg_ref, w2_ref, b2_ref,
             wp1_ref, bp1_ref, wp2_ref, bp2_ref, n_ref, nbf_ref):
    i = pl.program_id(0)
    agg = jnp.concatenate(
        [g0_ref[...], g1_ref[...], g2_ref[...], g3_ref[...]], axis=1)
    h1 = jnp.concatenate([h1t_ref[q] for q in range(4)], axis=1)
    u = (agg + h1) / (deg_ref[:, :1] + 1.0)
    h2 = lax.dot_general(u, w2_ref[...], (((1,), (1,)), ((), ())))
    h2 = jnp.maximum(h2 + b2_ref[...], 0.0)
    e = lax.dot_general(h2, wp1_ref[...], (((1,), (1,)), ((), ())))
    e = e + bp1_ref[...]
    e = jnp.where(e > 0.0, e, jnp.exp(e) - 1.0)
    p = lax.dot_general(e, wp2_ref[...], (((1,), (1,)), ((), ())))
    p = p + bp2_ref[...]
    nrm = jnp.sqrt(jnp.sum(p * p, axis=1, keepdims=True))
    n = p / jnp.maximum(nrm, 1e-12)
    row = i * BN + lax.broadcasted_iota(jnp.int32, (BN, 1), 0)
    n = jnp.where(row < N, n, 0.0)
    n_ref[...] = n
    nbf_ref[...] = n.astype(jnp.bfloat16)


def _loss_body(q_ref, k_ref, s_ref):
    j = pl.program_id(1)

    @pl.when(j == 0)
    def _():
        s_ref[...] = jnp.zeros_like(s_ref)

    z = lax.dot_general(q_ref[...], k_ref[...], (((1,), (1,)), ((), ())),
                        preferred_element_type=jnp.float32)
    s_ref[...] += jnp.sum(jnp.exp(z * INV_T), axis=1, keepdims=True)


def _fin_body(n1_ref, n2_ref, s11_ref, s12_ref, s21_ref, s22_ref, o_ref):
    i = pl.program_id(0)

    @pl.when(i == 0)
    def _():
        o_ref[...] = jnp.zeros_like(o_ref)

    n1 = n1_ref[...]
    n2 = n2_ref[...]
    d = jnp.sum(n1 * n2, axis=1, keepdims=True)
    q1 = jnp.sum(n1 * n1, axis=1, keepdims=True)
    q2 = jnp.sum(n2 * n2, axis=1, keepdims=True)
    den1 = (s11_ref[...] + s12_ref[...]) - float(2 * NPAD) - jnp.exp(q1 * INV_T)
    den2 = (s22_ref[...] + s21_ref[...]) - float(2 * NPAD) - jnp.exp(q2 * INV_T)
    v = -d * INV_T + 0.5 * (jnp.log(den1) + jnp.log(den2))
    row = i * BN + lax.broadcasted_iota(jnp.int32, (BN, 1), 0)
    v = jnp.where(row < N, v, 0.0)
    o_ref[...] = o_ref[...] + jnp.sum(v) / float(N)


_l1_call = pl.pallas_call(
    _l1_body,
    grid=(NP // BN,),
    in_specs=[
        pl.BlockSpec((BN, 128), lambda i: (i, 0)),
        pl.BlockSpec((BN, 128), lambda i: (i, 0)),
        pl.BlockSpec((BN, 256), lambda i: (i, 0)),
        pl.BlockSpec((BN, 128), lambda i: (i, 0)),
        pl.BlockSpec((512, 256), lambda i: (0, 0)),
        pl.BlockSpec((1, 512), lambda i: (0, 0)),
    ],
    out_specs=pl.BlockSpec((4, BN, 128), lambda i: (0, i, 0)),
    out_shape=jax.ShapeDtypeStruct((4, NP, 128), jnp.float32),
)

_l2_call = pl.pallas_call(
    _l2_body,
    grid=(NP // BN,),
    in_specs=[
        pl.BlockSpec((BN, 128), lambda i: (i, 0)),
        pl.BlockSpec((BN, 128), lambda i: (i, 0)),
        pl.BlockSpec((BN, 128), lambda i: (i, 0)),
        pl.BlockSpec((BN, 128), lambda i: (i, 0)),
        pl.BlockSpec((4, BN, 128), lambda i: (0, i, 0)),
        pl.BlockSpec((BN, 128), lambda i: (i, 0)),
        pl.BlockSpec((256, 512), lambda i: (0, 0)),
        pl.BlockSpec((1, 256), lambda i: (0, 0)),
        pl.BlockSpec((128, 256), lambda i: (0, 0)),
        pl.BlockSpec((1, 128), lambda i: (0, 0)),
        pl.BlockSpec((256, 128), lambda i: (0, 0)),
        pl.BlockSpec((1, 256), lambda i: (0, 0)),
    ],
    out_specs=[
        pl.BlockSpec((BN, 256), lambda i: (i, 0)),
        pl.BlockSpec((BN, 256), lambda i: (i, 0)),
    ],
    out_shape=[
        jax.ShapeDtypeStruct((NP, 256), jnp.float32),
        jax.ShapeDtypeStruct((NP, 256), jnp.bfloat16),
    ],
)

_loss_call = pl.pallas_call(
    _loss_body,
    grid=(NP // BI, NP // BJ),
    in_specs=[
        pl.BlockSpec((BI, 256), lambda i, j: (i, 0)),
        pl.BlockSpec((BJ, 256), lambda i, j: (j, 0)),
    ],
    out_specs=pl.BlockSpec((BI, 1), lambda i, j: (i, 0)),
    out_shape=jax.ShapeDtypeStruct((NP, 1), jnp.float32),
)

_fin_call = pl.pallas_call(
    _fin_body,
    grid=(NP // BN,),
    in_specs=[
        pl.BlockSpec((BN, 256), lambda i: (i, 0)),
        pl.BlockSpec((BN, 256), lambda i: (i, 0)),
        pl.BlockSpec((BN, 1), lambda i: (i, 0)),
        pl.BlockSpec((BN, 1), lambda i: (i, 0)),
        pl.BlockSpec((BN, 1), lambda i: (i, 0)),
        pl.BlockSpec((BN, 1), lambda i: (i, 0)),
    ],
    out_specs=pl.BlockSpec((1, 1), lambda i: (0, 0)),
    out_shape=jax.ShapeDtypeStruct((1, 1), jnp.float32),
)


@jax.jit
def kernel(features1, edge_index1, features2, edge_index2,
           W1, b1, W2, b2, Wp1, bp1, Wp2, bp2):
    x1 = jnp.pad(features1, ((0, NPAD), (0, 0)))
    x2 = jnp.pad(features2, ((0, NPAD), (0, 0)))
    src1 = edge_index1[0].astype(jnp.int32)
    dst1 = edge_index1[1].astype(jnp.int32)
    src2 = edge_index2[0].astype(jnp.int32)
    dst2 = edge_index2[1].astype(jnp.int32)
    z128 = jnp.zeros((NP, 128), jnp.float32)
    ones = jnp.ones((K, 128), jnp.float32)

    sc_agg1, sc_agg2 = _sc_kernels()
    a1a, a1b, a2a, a2b, deg1, deg2 = sc_agg1(
        x1[:, :128], x1[:, 128:], x2[:, :128], x2[:, 128:],
        src1, dst1, src2, dst2, z128, ones)

    h1t_1 = _l1_call(a1a, a1b, x1, deg1, W1, b1.reshape(1, -1))
    h1t_2 = _l1_call(a2a, a2b, x2, deg2, W1, b1.reshape(1, -1))

    g1 = sc_agg2(h1t_1[0], h1t_1[1], h1t_1[2], h1t_1[3], src1, dst1, z128)
    n1, n1bf = _l2_call(g1[0], g1[1], g1[2], g1[3], h1t_1, deg1,
                        W2, b2.reshape(1, -1),
                        Wp1, bp1.reshape(1, -1), Wp2, bp2.reshape(1, -1))
    s11 = _loss_call(n1bf, n1bf)

    g2 = sc_agg2(h1t_2[0], h1t_2[1], h1t_2[2], h1t_2[3], src2, dst2, z128)
    n2, n2bf = _l2_call(g2[0], g2[1], g2[2], g2[3], h1t_2, deg2,
                        W2, b2.reshape(1, -1),
                        Wp1, bp1.reshape(1, -1), Wp2, bp2.reshape(1, -1))
    s12 = _loss_call(n1bf, n2bf)
    s21 = _loss_call(n2bf, n1bf)
    s22 = _loss_call(n2bf, n2bf)

    out = _fin_call(n1, n2, s11, s12, s21, s22)
    return out[0, 0]
